# baseline (speedup 1.0000x reference)
import functools

import jax
import jax.numpy as jnp
from jax import lax
from jax.experimental import pallas as pl
from jax.experimental.pallas import tpu as pltpu
from jax.experimental.pallas import tpu_sc as plsc

_BATCH = 16384
_D = 64
_NC = 2
_NS = 16
_NW = _NC * _NS
_BPW = _BATCH // _NW
_CHUNK = 128
_NCH = _BPW // _CHUNK
_L = 16


def _tile_body(feat_hbm, lab_hbm, cent_hbm, out_hbm,
               idx_v, rows_v, feat_v, part_v, gsem, fsem):
    wid = lax.axis_index("s") * _NC + lax.axis_index("c")

    pltpu.sync_copy(lab_hbm.at[wid], idx_v)

    fcopy = pltpu.make_async_copy(feat_hbm.at[wid], feat_v, fsem)
    fcopy.start()
    gathers = []
    for j in range(_NCH):
        g = pltpu.make_async_copy(
            cent_hbm.at[idx_v.at[j]],
            rows_v.at[pl.ds(j * _CHUNK, _CHUNK)],
            gsem,
        )
        g.start()
        gathers.append(g)
    fcopy.wait()
    for g in gathers:
        g.wait()

    zero = jnp.zeros((_L,), jnp.float32)

    def body(r, accs):
        a0, a1, a2, a3 = accs
        d0 = feat_v[r, pl.ds(0, _L)] - rows_v[r, pl.ds(0, _L)]
        d1 = feat_v[r, pl.ds(_L, _L)] - rows_v[r, pl.ds(_L, _L)]
        d2 = feat_v[r, pl.ds(2 * _L, _L)] - rows_v[r, pl.ds(2 * _L, _L)]
        d3 = feat_v[r, pl.ds(3 * _L, _L)] - rows_v[r, pl.ds(3 * _L, _L)]
        return (a0 + d0 * d0, a1 + d1 * d1, a2 + d2 * d2, a3 + d3 * d3)

    a0, a1, a2, a3 = lax.fori_loop(0, _BPW, body, (zero, zero, zero, zero))
    part_v[...] = (a0 + a1) + (a2 + a3)
    pltpu.sync_copy(part_v, out_hbm.at[wid])


_mesh = plsc.VectorSubcoreMesh(core_axis_name="c", subcore_axis_name="s")

_sc_call = functools.partial(
    pl.kernel,
    mesh=_mesh,
    out_type=jax.ShapeDtypeStruct((_NW, _L), jnp.float32),
    scratch_types=[
        pltpu.VMEM((_NCH, _CHUNK), jnp.int32),
        pltpu.VMEM((_BPW, _D), jnp.float32),
        pltpu.VMEM((_BPW, _D), jnp.float32),
        pltpu.VMEM((_L,), jnp.float32),
        pltpu.SemaphoreType.DMA,
        pltpu.SemaphoreType.DMA,
    ],
    compiler_params=pltpu.CompilerParams(use_tc_tiling_on_sc=False),
)(_tile_body)


def kernel(features, labels, centers):
    feat = features.reshape(_NW, _BPW, _D)
    lab = labels.astype(jnp.int32).reshape(_NW, _NCH, _CHUNK)
    partials = _sc_call(feat, lab, centers)
    return jnp.sum(partials) * 0.5


# speedup vs baseline: 1.2609x; 1.2609x over previous
import functools

import jax
import jax.numpy as jnp
from jax import lax
from jax.experimental import pallas as pl
from jax.experimental.pallas import tpu as pltpu
from jax.experimental.pallas import tpu_sc as plsc

_BATCH = 16384
_D = 64
_NC = 2
_NS = 16
_NW = _NC * _NS
_BPW = _BATCH // _NW
_CH = 128
_NCH = _BPW // _CH
_L = 16


def _tile_body(feat_hbm, lab_hbm, cent_hbm, out_hbm,
               lab_v, lab_s, rows_c, feat_c, part_v, gsem, fsem, lsem):
    wid = lax.axis_index("s") * _NC + lax.axis_index("c")
    base = wid * _BPW

    pltpu.sync_copy(lab_hbm.at[pl.ds(base, _BPW)], lab_v)

    zero = jnp.zeros((_L,), jnp.float32)
    accs = (zero, zero, zero, zero)

    for c in range(_NCH):
        cbase = base + c * _CH

        fcopy = pltpu.make_async_copy(
            feat_hbm.at[pl.ds(cbase, _CH)], feat_c, fsem)
        fcopy.start()

        def enq(i, _, c=c):
            v = lab_v[pl.ds(c * _CH + i * _L, _L)]
            for j in range(_L):
                pltpu.make_async_copy(
                    cent_hbm.at[v[j]], rows_c.at[i * _L + j], gsem).start()
            return 0

        lax.fori_loop(0, _CH // _L, enq, 0)

        for i in range(_CH):
            pltpu.make_async_copy(
                cent_hbm.at[0], rows_c.at[i], gsem).wait()
        fcopy.wait()

        def body(r, accs):
            a0, a1, a2, a3 = accs
            d0 = feat_c[r, pl.ds(0, _L)] - rows_c[r, pl.ds(0, _L)]
            d1 = feat_c[r, pl.ds(_L, _L)] - rows_c[r, pl.ds(_L, _L)]
            d2 = feat_c[r, pl.ds(2 * _L, _L)] - rows_c[r, pl.ds(2 * _L, _L)]
            d3 = feat_c[r, pl.ds(3 * _L, _L)] - rows_c[r, pl.ds(3 * _L, _L)]
            return (a0 + d0 * d0, a1 + d1 * d1, a2 + d2 * d2, a3 + d3 * d3)

        accs = lax.fori_loop(0, _CH, body, accs)

    a0, a1, a2, a3 = accs
    part_v[...] = (a0 + a1) + (a2 + a3)
    pltpu.sync_copy(part_v, out_hbm.at[pl.ds(wid * _L, _L)])


_mesh = plsc.VectorSubcoreMesh(core_axis_name="c", subcore_axis_name="s")

_sc_call = functools.partial(
    pl.kernel,
    mesh=_mesh,
    out_type=jax.ShapeDtypeStruct((_NW * _L,), jnp.float32),
    scratch_types=[
        pltpu.VMEM((_BPW,), jnp.int32),
        pltpu.SMEM((_BPW,), jnp.int32),
        pltpu.VMEM((_CH, _D), jnp.float32),
        pltpu.VMEM((_CH, _D), jnp.float32),
        pltpu.VMEM((_L,), jnp.float32),
        pltpu.SemaphoreType.DMA,
        pltpu.SemaphoreType.DMA,
        pltpu.SemaphoreType.DMA,
    ],
    compiler_params=pltpu.CompilerParams(use_tc_tiling_on_sc=True),
)(_tile_body)


def kernel(features, labels, centers):
    lab = labels.astype(jnp.int32)
    partials = _sc_call(features, lab, centers)
    return jnp.sum(partials) * 0.5
